# baseline (speedup 1.0000x reference)
import functools

import jax
import jax.numpy as jnp
from jax import lax
from jax.experimental import pallas as pl
from jax.experimental.pallas import tpu as pltpu
from jax.experimental.pallas import tpu_sc as plsc

_N = 512
_D = 512
_NC = 2
_NS = 16
_L = 16
_ROWS = _N // _NS


def _lower_bound(k_v, E, val):
    def body(_, lohi):
        lo, hi = lohi
        mid = lax.shift_right_logical(lo + hi, 1)
        pred = k_v[pl.ds(mid, _L)][0] < val
        return (jnp.where(pred, mid + 1, lo), jnp.where(pred, hi, mid))

    nbits = max(1, (E - 1).bit_length())
    lo, _ = lax.fori_loop(0, nbits, body, (jnp.int32(0), jnp.int32(E)))
    return lo


def _sc_scatter(kk, ww, BC, E, g0):
    gpc = BC // _NC
    mesh = plsc.VectorSubcoreMesh(core_axis_name="c", subcore_axis_name="s")

    @functools.partial(
        pl.kernel,
        out_type=jax.ShapeDtypeStruct((BC, _N, _N), jnp.float32),
        mesh=mesh,
        compiler_params=pltpu.CompilerParams(needs_layout_passes=False),
        scratch_types=[
            pltpu.VMEM((E + _L,), jnp.int32),
            pltpu.VMEM((E,), jnp.float32),
            pltpu.VMEM((E + _L,), jnp.int32),
            pltpu.VMEM((E,), jnp.float32),
            pltpu.VMEM((_ROWS, _N), jnp.float32),
            pltpu.SemaphoreType.DMA,
        ],
    )
    def k(kk_h, ww_h, pre_h, k1_v, w1_v, k2_v, w2_v, buf, sem):
        c = lax.axis_index("c")
        t = lax.axis_index("s")
        lo = t * _ROWS
        hi = lo + _ROWS
        zeros16 = jnp.zeros((_L,), jnp.float32)
        sent16 = jnp.full((_L,), -1, jnp.int32)

        def per_graph(g, carry):
            b = c * gpc + g
            bg = g0 + b
            e0 = bg * E
            cps = [
                pltpu.async_copy(kk_h.at[0, pl.ds(e0, E)], k1_v.at[pl.ds(0, E)], sem),
                pltpu.async_copy(ww_h.at[0, pl.ds(e0, E)], w1_v, sem),
                pltpu.async_copy(kk_h.at[1, pl.ds(e0, E)], k2_v.at[pl.ds(0, E)], sem),
                pltpu.async_copy(ww_h.at[1, pl.ds(e0, E)], w2_v, sem),
            ]

            @plsc.parallel_loop(0, _ROWS * _N // _L, unroll=8)
            def _zero(i):
                r = lax.shift_right_logical(i, 5)
                off = jnp.bitwise_and(i, 31) * _L
                buf[r, pl.ds(off, _L)] = zeros16

            for cp in cps:
                cp.wait()
            k1_v[pl.ds(E, _L)] = sent16
            k2_v[pl.ds(E, _L)] = sent16

            base = bg * (_N * _N)
            lim_lo = base + lo * _N
            lim_hi = base + hi * _N

            def scan_pass(k_v, w_v):
                c0 = lax.shift_right_logical(_lower_bound(k_v, E, lim_lo), 4)
                e1 = _lower_bound(k_v, E, lim_hi)
                c1 = lax.shift_right_logical(e1 + _L - 1, 4)

                def scat(i, carry2):
                    key = k_v[pl.ds(i * _L, _L)]
                    nxt = k_v[pl.ds(i * _L + 1, _L)]
                    wv = w_v[pl.ds(i * _L, _L)]
                    r = jnp.bitwise_and(lax.shift_right_logical(key, 9), _N - 1)
                    col = jnp.bitwise_and(key, _N - 1)
                    m = (key >= lim_lo) & (key < lim_hi) & (key != nxt)
                    plsc.store_scatter(buf, [r - lo, col], wv, mask=m)
                    return carry2

                lax.fori_loop(c0, c1, scat, 0)

            scan_pass(k1_v, w1_v)
            scan_pass(k2_v, w2_v)
            pltpu.sync_copy(buf, pre_h.at[b, pl.ds(lo, _ROWS), :])
            return carry

        lax.fori_loop(0, gpc, per_graph, 0)

    return k(kk, ww)


def _mm_body(lens_ref, pre_ref, w_ref, bias_ref, out_ref):
    b = pl.program_id(0)
    n = lens_ref[b]
    acc = lax.dot_general(
        pre_ref[0],
        w_ref[...],
        dimension_numbers=(((1,), (1,)), ((), ())),
        preferred_element_type=jnp.float32,
    )
    rows = lax.broadcasted_iota(jnp.int32, (_N, _D), 0)
    out_ref[0] = jnp.where(rows < n, acc + bias_ref[...][None, :], 0.0)


def _matmul_mask(pre, graph_lens, W, b):
    B = pre.shape[0]
    return pl.pallas_call(
        _mm_body,
        grid=(B,),
        in_specs=[
            pl.BlockSpec(memory_space=pltpu.SMEM),
            pl.BlockSpec((1, _N, _N), lambda i: (i, 0, 0)),
            pl.BlockSpec((_D, _N), lambda i: (0, 0)),
            pl.BlockSpec((_D,), lambda i: (0,)),
        ],
        out_specs=pl.BlockSpec((1, _N, _D), lambda i: (i, 0, 0)),
        out_shape=jax.ShapeDtypeStruct((B, _N, _D), jnp.float32),
    )(graph_lens.astype(jnp.int32), pre, W, b)


def kernel(edge_index, edge_weight, graph_lens, W, b):
    B, E, _ = edge_index.shape
    src = edge_index[..., 0].astype(jnp.int32)
    dst = edge_index[..., 1].astype(jnp.int32)
    w = edge_weight.astype(jnp.float32)
    base = jnp.arange(B, dtype=jnp.int32)[:, None] * (_N * _N)
    key1 = (base + src * _N + dst).ravel()
    key2 = (base + dst * _N + src).ravel()
    wf = w.ravel()
    keys = jnp.stack([key1, key2], axis=0)
    vals = jnp.stack([wf, wf], axis=0)
    kk, ww = lax.sort((keys, vals), dimension=1, num_keys=1, is_stable=False)
    pre = _sc_scatter(kk, ww, B, E, 0)
    return _matmul_mask(pre, graph_lens, W, b)


# speedup vs baseline: 2.7086x; 2.7086x over previous
import functools

import jax
import jax.numpy as jnp
from jax import lax
from jax.experimental import pallas as pl
from jax.experimental.pallas import tpu as pltpu
from jax.experimental.pallas import tpu_sc as plsc

_N = 512
_D = 512
_NC = 2
_NS = 16
_L = 16
_ROWS = _N // _NS


def _lower_bound(k_v, E, val):
    def body(_, lohi):
        lo, hi = lohi
        mid = lax.shift_right_logical(lo + hi, 1)
        pred = k_v[pl.ds(mid, _L)][0] < val
        return (jnp.where(pred, mid + 1, lo), jnp.where(pred, hi, mid))

    nbits = max(1, (E - 1).bit_length())
    lo, _ = lax.fori_loop(0, nbits, body, (jnp.int32(0), jnp.int32(E)))
    return lo


def _sc_scatter(kk, ww, lens, B, E):
    gpc = B // _NC
    mesh = plsc.VectorSubcoreMesh(core_axis_name="c", subcore_axis_name="s")

    @functools.partial(
        pl.kernel,
        out_type=jax.ShapeDtypeStruct((B, _N, _N), jnp.float32),
        mesh=mesh,
        compiler_params=pltpu.CompilerParams(needs_layout_passes=False),
        scratch_types=[
            pltpu.VMEM((E + _L,), jnp.int32),
            pltpu.VMEM((E,), jnp.float32),
            pltpu.VMEM((E + _L,), jnp.int32),
            pltpu.VMEM((E,), jnp.float32),
            pltpu.VMEM((_ROWS, _N), jnp.float32),
            pltpu.VMEM((E + _L,), jnp.int32),
            pltpu.VMEM((E,), jnp.float32),
            pltpu.VMEM((E + _L,), jnp.int32),
            pltpu.VMEM((E,), jnp.float32),
            pltpu.VMEM((_ROWS, _N), jnp.float32),
            pltpu.VMEM((B + _L,), jnp.int32),
            pltpu.SemaphoreType.DMA,
            pltpu.SemaphoreType.DMA,
            pltpu.SemaphoreType.DMA,
        ],
    )
    def k(kk_h, ww_h, lens_h, pre_h,
          k1a, w1a, k2a, w2a, bufa, k1b, w1b, k2b, w2b, bufb, lens_v,
          sem_i, sem_oa, sem_ob):
        c = lax.axis_index("c")
        t = lax.axis_index("s")
        lo = t * _ROWS
        hi = lo + _ROWS
        zeros16 = jnp.zeros((_L,), jnp.float32)
        sent16 = jnp.full((_L,), -1, jnp.int32)
        bufs_a = (k1a, w1a, k2a, w2a, bufa, sem_oa)
        bufs_b = (k1b, w1b, k2b, w2b, bufb, sem_ob)

        pltpu.sync_copy(lens_h, lens_v.at[pl.ds(0, B)])

        def in_copies(g, bufs):
            k1_v, w1_v, k2_v, w2_v, _, _ = bufs
            e0 = (c * gpc + g) * E
            return [
                pltpu.make_async_copy(
                    kk_h.at[0, pl.ds(e0, E)], k1_v.at[pl.ds(0, E)], sem_i),
                pltpu.make_async_copy(ww_h.at[0, pl.ds(e0, E)], w1_v, sem_i),
                pltpu.make_async_copy(
                    kk_h.at[1, pl.ds(e0, E)], k2_v.at[pl.ds(0, E)], sem_i),
                pltpu.make_async_copy(ww_h.at[1, pl.ds(e0, E)], w2_v, sem_i),
            ]

        def out_copy(bg, bufs):
            return pltpu.make_async_copy(
                bufs[4], pre_h.at[bg, pl.ds(lo, _ROWS), :], bufs[5])

        def len_of(bg):
            return lens_v[pl.ds(bg, _L)][0]

        def process(g, bufs, next_bufs, issue_next):
            k1_v, w1_v, k2_v, w2_v, buf, _ = bufs
            bg = c * gpc + g

            @pl.when((g >= 2) & (lo < len_of(bg - 2)))
            def _():
                out_copy(bg - 2, bufs).wait()

            for cp in in_copies(g, bufs):
                cp.wait()
            k1_v[pl.ds(E, _L)] = sent16
            k2_v[pl.ds(E, _L)] = sent16

            @pl.when(issue_next)
            def _():
                for cp in in_copies(g + 1, next_bufs):
                    cp.start()

            len_b = len_of(bg)

            @pl.when(lo < len_b)
            def _():
                @plsc.parallel_loop(0, _ROWS * _N // _L, unroll=8)
                def _zero(i):
                    r = lax.shift_right_logical(i, 5)
                    off = jnp.bitwise_and(i, 31) * _L
                    buf[r, pl.ds(off, _L)] = zeros16

                base = bg * (_N * _N)
                lim_lo = base + lo * _N
                lim_hi = base + jnp.minimum(hi, len_b) * _N

                def scan_pass(k_v, w_v):
                    c0 = lax.shift_right_logical(
                        _lower_bound(k_v, E, lim_lo), 4)
                    e1 = _lower_bound(k_v, E, lim_hi)
                    c1 = lax.shift_right_logical(e1 + _L - 1, 4)

                    def scat(i, carry2):
                        key = k_v[pl.ds(i * _L, _L)]
                        nxt = k_v[pl.ds(i * _L + 1, _L)]
                        wv = w_v[pl.ds(i * _L, _L)]
                        r = jnp.bitwise_and(
                            lax.shift_right_logical(key, 9), _N - 1)
                        col = jnp.bitwise_and(key, _N - 1)
                        m = (key >= lim_lo) & (key < lim_hi) & (key != nxt)
                        plsc.store_scatter(buf, [r - lo, col], wv, mask=m)
                        return carry2

                    lax.fori_loop(c0, c1, scat, 0)

                scan_pass(k1_v, w1_v)
                scan_pass(k2_v, w2_v)
                out_copy(bg, bufs).start()

        for cp in in_copies(0, bufs_a):
            cp.start()

        def per_pair(g2, carry):
            g = g2 * 2
            process(g, bufs_a, bufs_b, g + 1 < gpc)
            process(g + 1, bufs_b, bufs_a, g + 2 < gpc)
            return carry

        lax.fori_loop(0, gpc // 2, per_pair, 0)

        for gl, bufs in ((gpc - 2, bufs_a), (gpc - 1, bufs_b)):
            bgl = c * gpc + gl

            @pl.when(lo < len_of(bgl))
            def _():
                out_copy(bgl, bufs).wait()

    return k(kk, ww, lens)


def _mm_body(lens_ref, pre_ref, w_ref, bias_ref, out_ref):
    b = pl.program_id(0)
    n = lens_ref[b]
    acc = lax.dot_general(
        pre_ref[0],
        w_ref[...],
        dimension_numbers=(((1,), (1,)), ((), ())),
        preferred_element_type=jnp.float32,
    )
    rows = lax.broadcasted_iota(jnp.int32, (_N, _D), 0)
    out_ref[0] = jnp.where(rows < n, acc + bias_ref[...][None, :], 0.0)


def _matmul_mask(pre, graph_lens, W, b):
    B = pre.shape[0]
    return pl.pallas_call(
        _mm_body,
        grid=(B,),
        in_specs=[
            pl.BlockSpec(memory_space=pltpu.SMEM),
            pl.BlockSpec((1, _N, _N), lambda i: (i, 0, 0)),
            pl.BlockSpec((_D, _N), lambda i: (0, 0)),
            pl.BlockSpec((_D,), lambda i: (0,)),
        ],
        out_specs=pl.BlockSpec((1, _N, _D), lambda i: (i, 0, 0)),
        out_shape=jax.ShapeDtypeStruct((B, _N, _D), jnp.float32),
    )(graph_lens.astype(jnp.int32), pre, W, b)


def kernel(edge_index, edge_weight, graph_lens, W, b):
    B, E, _ = edge_index.shape
    src = edge_index[..., 0].astype(jnp.int32)
    dst = edge_index[..., 1].astype(jnp.int32)
    w = edge_weight.astype(jnp.float32)
    base = jnp.arange(B, dtype=jnp.int32)[:, None] * (_N * _N)
    key1 = (base + src * _N + dst).ravel()
    key2 = (base + dst * _N + src).ravel()
    wf = w.ravel()
    k1, w1 = lax.sort((key1, wf), dimension=0, num_keys=1, is_stable=False)
    k2, w2 = lax.sort((key2, wf), dimension=0, num_keys=1, is_stable=False)
    kk = jnp.stack([k1, k2], axis=0)
    ww = jnp.stack([w1, w2], axis=0)
    pre = _sc_scatter(kk, ww, graph_lens.astype(jnp.int32), B, E)
    return _matmul_mask(pre, graph_lens, W, b)
